# baseline (speedup 1.0000x reference)
import dataclasses
import functools

import jax
import jax.numpy as jnp
from jax import lax
from jax.experimental import pallas as pl
from jax.experimental.pallas import tpu as pltpu
from jax.experimental.pallas import tpu_sc as plsc

N = 10000
D = 512
S = 10
L1 = 512
L2 = 256
DEC1 = 512
DEC2 = 512
B = 1024

SP = 16
NC = 2
NS = 16
NW = NC * NS
SEEDS_W = B // NW
P = B * S
P_W = P // NW
PC = 64

_sc_mesh = plsc.VectorSubcoreMesh(core_axis_name="c", subcore_axis_name="s")
_sc_params = pltpu.CompilerParams(use_tc_tiling_on_sc=False)
if "needs_layout_passes" in pltpu.CompilerParams.__dataclass_fields__:
    _sc_params = dataclasses.replace(_sc_params, needs_layout_passes=False)


@functools.partial(
    pl.kernel,
    mesh=_sc_mesh,
    compiler_params=_sc_params,
    out_type=[
        jax.ShapeDtypeStruct((P, D), jnp.bfloat16),
        jax.ShapeDtypeStruct((B, D), jnp.bfloat16),
    ],
    scratch_types=[
        pltpu.VMEM((SEEDS_W,), jnp.int32),
        pltpu.VMEM((SEEDS_W, SP), jnp.int32),
        pltpu.VMEM((SEEDS_W * SP, SP), jnp.int32),
        pltpu.VMEM((S, P_W), jnp.int32),
        pltpu.VMEM((PC, D), jnp.bfloat16),
        pltpu.VMEM((PC, D), jnp.bfloat16),
        pltpu.VMEM((PC, D), jnp.bfloat16),
        pltpu.SemaphoreType.DMA,
    ],
)
def _sc_gather_agg(x_hbm, neigh_hbm, nodes_hbm, agg_hbm, orig_hbm,
                   sv, nbv, nb2v, nb2t, acc, ra, rb, sem):
    wid = lax.axis_index("s") * NC + lax.axis_index("c")
    base = wid * P_W

    pltpu.sync_copy(nodes_hbm.at[pl.ds(wid * SEEDS_W, SEEDS_W)], sv)
    pltpu.async_copy(neigh_hbm.at[sv], nbv, sem).wait()

    for g in range(0, SEEDS_W, 8):
        hs = [
            pltpu.async_copy(
                neigh_hbm.at[nbv.at[i]], nb2v.at[pl.ds(i * SP, SP)], sem
            )
            for i in range(g, g + 8)
        ]
        for h in hs:
            h.wait()

    pltpu.async_copy(x_hbm.at[sv], ra.at[pl.ds(0, SEEDS_W)], sem).wait()
    pltpu.sync_copy(
        ra.at[pl.ds(0, SEEDS_W)], orig_hbm.at[pl.ds(wid * SEEDS_W, SEEDS_W)]
    )

    for k in range(S):
        kvec = jnp.full((16,), k, jnp.int32)

        @pl.loop(0, P_W, step=16)
        def _compact(c):
            pvec = lax.iota(jnp.int32, 16) + c
            i = pvec // S
            srow = i * SP + (pvec - i * S)
            vals = plsc.load_gather(nb2v, [srow, kvec])
            nb2t[k, pl.ds(c, 16)] = vals

    def fire(k, j, buf):
        return pltpu.async_copy(
            x_hbm.at[nb2t.at[k, pl.ds(j * PC, PC)]], buf, sem
        )

    def acc_rows(buf, first):
        @pl.loop(0, PC)
        def _row(r):
            for c in range(0, D, 32):
                if first:
                    acc[r, pl.ds(c, 32)] = buf[r, pl.ds(c, 32)]
                else:
                    plsc.addupdate(acc.at[r, pl.ds(c, 32)], buf[r, pl.ds(c, 32)])

    hout = None
    for j in range(P_W // PC):
        bufs = (ra, rb)
        h = [None] * S
        h[0] = fire(0, j, ra)
        h[1] = fire(1, j, rb)
        h[0].wait()
        if hout is not None:
            hout.wait()
        acc_rows(ra, True)
        for k in range(2, S + 1):
            if k <= S - 1:
                h[k] = fire(k, j, bufs[k % 2])
            h[k - 1].wait()
            acc_rows(bufs[(k - 1) % 2], False)
        hout = pltpu.async_copy(
            acc, agg_hbm.at[pl.ds(base + j * PC, PC)], sem
        )
    hout.wait()


def _enc1_body(agg_ref, w1_ref, m2_ref):
    w1b = w1_ref[...].astype(jnp.bfloat16)
    z = jnp.dot(agg_ref[...], w1b, preferred_element_type=jnp.float32)
    h = jnp.maximum(z * (1.0 / S), 0.0)
    m2_ref[...] = jnp.mean(h.reshape(h.shape[0] // S, S, L1), axis=1)


def _norm_rows(v):
    n = jnp.sqrt(jnp.sum(v * v, axis=-1, keepdims=True))
    return v / jnp.maximum(n, 1e-12)


def _dense_body(m2_ref, orig_ref, adj_ref, w2_ref, w3_ref, d1w_ref, d1b_ref,
                d2w_ref, d2b_ref, m1w_ref, m2w_ref, dw_ref, db_ref, dlw_ref,
                mu_ref, lvs_ref, rec_ref, pred_ref):
    f32 = jnp.float32
    m2v = m2_ref[...]
    mu = jnp.maximum(jnp.dot(m2v, w2_ref[...], preferred_element_type=f32), 0.0)
    lv = jnp.maximum(jnp.dot(m2v, w3_ref[...], preferred_element_type=f32), 0.0)
    mu_ref[...] = mu
    lvs_ref[...] = -lv
    h = _norm_rows(mu)
    o = jnp.maximum(
        jnp.dot(h, d1w_ref[...], preferred_element_type=f32) + d1b_ref[...], 0.0
    )
    o = jnp.dot(o, d2w_ref[...], preferred_element_type=f32) + d2b_ref[...]
    e1 = _norm_rows(jnp.dot(o, m1w_ref[...], preferred_element_type=f32))
    e2 = _norm_rows(jnp.dot(o, m2w_ref[...], preferred_element_type=f32))
    rec = lax.dot_general(e1, e2, (((1,), (1,)), ((), ())),
                          preferred_element_type=f32)
    rec_ref[...] = rec
    sup = jnp.dot(orig_ref[...], dw_ref[...].astype(jnp.bfloat16),
                  preferred_element_type=f32)
    dlw_row = dlw_ref[...].reshape(1, DEC2)
    t1 = jnp.dot(adj_ref[...], sup, preferred_element_type=f32) + db_ref[...]
    pred_ref[pl.ds(0, B), :] = jnp.sum(t1 * dlw_row, axis=1, keepdims=True)
    t2 = jnp.dot(rec, sup, preferred_element_type=f32) + db_ref[...]
    pred_ref[pl.ds(B, B), :] = jnp.sum(t2 * dlw_row, axis=1, keepdims=True)


def kernel(nodes, sub_adj, x, neigh, W1, W2, W3, dec1_W, dec1_b, dec2_W,
           dec2_b, map1_W, map2_W, disc_W, disc_b, disc_lin_W):
    f32 = jnp.float32
    neigh_pad = jnp.pad(neigh, ((0, 0), (0, SP - S)))
    agg, orig = _sc_gather_agg(x.astype(jnp.bfloat16), neigh_pad, nodes)

    GB = 8
    m2 = pl.pallas_call(
        _enc1_body,
        grid=(GB,),
        in_specs=[
            pl.BlockSpec((P // GB, D), lambda i: (i, 0)),
            pl.BlockSpec((D, L1), lambda i: (0, 0)),
        ],
        out_specs=pl.BlockSpec((B // GB, L1), lambda i: (i, 0)),
        out_shape=jax.ShapeDtypeStruct((B, L1), f32),
    )(agg, W1)

    mu, lvs, rec, pred = pl.pallas_call(
        _dense_body,
        out_shape=[
            jax.ShapeDtypeStruct((B, L2), f32),
            jax.ShapeDtypeStruct((B, L2), f32),
            jax.ShapeDtypeStruct((B, B), f32),
            jax.ShapeDtypeStruct((2 * B, 1), f32),
        ],
    )(m2, orig, sub_adj, W2, W3, dec1_W, dec1_b.reshape(1, DEC1), dec2_W,
      dec2_b.reshape(1, DEC2), map1_W, map2_W, disc_W, disc_b.reshape(1, DEC2),
      disc_lin_W)

    label = jnp.concatenate(
        [jnp.ones((B, 1), f32), jnp.zeros((B, 1), f32)], axis=0
    )
    return (mu, lvs, rec, pred, label)


# speedup vs baseline: 1.0084x; 1.0084x over previous
import dataclasses
import functools

import jax
import jax.numpy as jnp
from jax import lax
from jax.experimental import pallas as pl
from jax.experimental.pallas import tpu as pltpu
from jax.experimental.pallas import tpu_sc as plsc

N = 10000
D = 512
S = 10
L1 = 512
L2 = 256
DEC1 = 512
DEC2 = 512
B = 1024

SP = 16
NC = 2
NS = 16
NW = NC * NS
SEEDS_W = B // NW
P = B * S
P_W = P // NW
PC = 64

_sc_mesh = plsc.VectorSubcoreMesh(core_axis_name="c", subcore_axis_name="s")
_sc_params = pltpu.CompilerParams(use_tc_tiling_on_sc=False)
if "needs_layout_passes" in pltpu.CompilerParams.__dataclass_fields__:
    _sc_params = dataclasses.replace(_sc_params, needs_layout_passes=False)


@functools.partial(
    pl.kernel,
    mesh=_sc_mesh,
    compiler_params=_sc_params,
    out_type=[
        jax.ShapeDtypeStruct((P, D), jnp.float32),
        jax.ShapeDtypeStruct((B, D), jnp.float32),
    ],
    scratch_types=[
        pltpu.VMEM((SEEDS_W,), jnp.int32),
        pltpu.VMEM((SEEDS_W, SP), jnp.int32),
        pltpu.VMEM((SEEDS_W * SP, SP), jnp.int32),
        pltpu.VMEM((S, P_W), jnp.int32),
        pltpu.VMEM((PC, D), jnp.float32),
        pltpu.VMEM((PC, D), jnp.float32),
        pltpu.VMEM((PC, D), jnp.float32),
        pltpu.SemaphoreType.DMA,
    ],
)
def _sc_gather_agg(x_hbm, neigh_hbm, nodes_hbm, agg_hbm, orig_hbm,
                   sv, nbv, nb2v, nb2t, acc, ra, rb, sem):
    wid = lax.axis_index("s") * NC + lax.axis_index("c")
    base = wid * P_W

    pltpu.sync_copy(nodes_hbm.at[pl.ds(wid * SEEDS_W, SEEDS_W)], sv)
    pltpu.async_copy(neigh_hbm.at[sv], nbv, sem).wait()

    for g in range(0, SEEDS_W, 8):
        hs = [
            pltpu.async_copy(
                neigh_hbm.at[nbv.at[i]], nb2v.at[pl.ds(i * SP, SP)], sem
            )
            for i in range(g, g + 8)
        ]
        for h in hs:
            h.wait()

    pltpu.async_copy(x_hbm.at[sv], ra.at[pl.ds(0, SEEDS_W)], sem).wait()
    pltpu.sync_copy(
        ra.at[pl.ds(0, SEEDS_W)], orig_hbm.at[pl.ds(wid * SEEDS_W, SEEDS_W)]
    )

    for k in range(S):
        kvec = jnp.full((16,), k, jnp.int32)

        @pl.loop(0, P_W, step=16)
        def _compact(c):
            pvec = lax.iota(jnp.int32, 16) + c
            i = pvec // S
            srow = i * SP + (pvec - i * S)
            vals = plsc.load_gather(nb2v, [srow, kvec])
            nb2t[k, pl.ds(c, 16)] = vals

    def fire(k, j, buf):
        return pltpu.async_copy(
            x_hbm.at[nb2t.at[k, pl.ds(j * PC, PC)]], buf, sem
        )

    def acc_rows(buf, first):
        @pl.loop(0, PC)
        def _row(r):
            for c in range(0, D, 16):
                if first:
                    acc[r, pl.ds(c, 16)] = buf[r, pl.ds(c, 16)]
                else:
                    plsc.addupdate(acc.at[r, pl.ds(c, 16)], buf[r, pl.ds(c, 16)])

    hout = None
    for j in range(P_W // PC):
        bufs = (ra, rb)
        h = [None] * S
        h[0] = fire(0, j, ra)
        h[1] = fire(1, j, rb)
        h[0].wait()
        if hout is not None:
            hout.wait()
        acc_rows(ra, True)
        for k in range(2, S + 1):
            if k <= S - 1:
                h[k] = fire(k, j, bufs[k % 2])
            h[k - 1].wait()
            acc_rows(bufs[(k - 1) % 2], False)
        hout = pltpu.async_copy(
            acc, agg_hbm.at[pl.ds(base + j * PC, PC)], sem
        )
    hout.wait()


def _enc1_body(agg_ref, w1_ref, m2_ref):
    a = agg_ref[...] * (1.0 / S)
    h = jnp.maximum(
        jnp.dot(a, w1_ref[...], preferred_element_type=jnp.float32), 0.0
    )
    m2_ref[...] = jnp.mean(h.reshape(h.shape[0] // S, S, L1), axis=1)


def _norm_rows(v):
    n = jnp.sqrt(jnp.sum(v * v, axis=-1, keepdims=True))
    return v / jnp.maximum(n, 1e-12)


def _dense_body(m2_ref, orig_ref, adj_ref, w2_ref, w3_ref, d1w_ref, d1b_ref,
                d2w_ref, d2b_ref, m1w_ref, m2w_ref, dw_ref, db_ref, dlw_ref,
                mu_ref, lvs_ref, rec_ref, pred_ref):
    f32 = jnp.float32
    m2v = m2_ref[...]
    mu = jnp.maximum(jnp.dot(m2v, w2_ref[...], preferred_element_type=f32), 0.0)
    lv = jnp.maximum(jnp.dot(m2v, w3_ref[...], preferred_element_type=f32), 0.0)
    mu_ref[...] = mu
    lvs_ref[...] = -lv
    h = _norm_rows(mu)
    o = jnp.maximum(
        jnp.dot(h, d1w_ref[...], preferred_element_type=f32) + d1b_ref[...], 0.0
    )
    o = jnp.dot(o, d2w_ref[...], preferred_element_type=f32) + d2b_ref[...]
    e1 = _norm_rows(jnp.dot(o, m1w_ref[...], preferred_element_type=f32))
    e2 = _norm_rows(jnp.dot(o, m2w_ref[...], preferred_element_type=f32))
    rec = lax.dot_general(e1, e2, (((1,), (1,)), ((), ())),
                          preferred_element_type=f32)
    rec_ref[...] = rec
    sup = jnp.dot(orig_ref[...], dw_ref[...], preferred_element_type=f32)
    dlw_row = dlw_ref[...].reshape(1, DEC2)
    t1 = jnp.dot(adj_ref[...], sup, preferred_element_type=f32) + db_ref[...]
    pred_ref[pl.ds(0, B), :] = jnp.sum(t1 * dlw_row, axis=1, keepdims=True)
    t2 = jnp.dot(rec, sup, preferred_element_type=f32) + db_ref[...]
    pred_ref[pl.ds(B, B), :] = jnp.sum(t2 * dlw_row, axis=1, keepdims=True)


def kernel(nodes, sub_adj, x, neigh, W1, W2, W3, dec1_W, dec1_b, dec2_W,
           dec2_b, map1_W, map2_W, disc_W, disc_b, disc_lin_W):
    f32 = jnp.float32
    neigh_pad = jnp.pad(neigh, ((0, 0), (0, SP - S)))
    agg, orig = _sc_gather_agg(x, neigh_pad, nodes)

    GB = 8
    m2 = pl.pallas_call(
        _enc1_body,
        grid=(GB,),
        in_specs=[
            pl.BlockSpec((P // GB, D), lambda i: (i, 0)),
            pl.BlockSpec((D, L1), lambda i: (0, 0)),
        ],
        out_specs=pl.BlockSpec((B // GB, L1), lambda i: (i, 0)),
        out_shape=jax.ShapeDtypeStruct((B, L1), f32),
    )(agg, W1)

    mu, lvs, rec, pred = pl.pallas_call(
        _dense_body,
        out_shape=[
            jax.ShapeDtypeStruct((B, L2), f32),
            jax.ShapeDtypeStruct((B, L2), f32),
            jax.ShapeDtypeStruct((B, B), f32),
            jax.ShapeDtypeStruct((2 * B, 1), f32),
        ],
    )(m2, orig, sub_adj, W2, W3, dec1_W, dec1_b.reshape(1, DEC1), dec2_W,
      dec2_b.reshape(1, DEC2), map1_W, map2_W, disc_W, disc_b.reshape(1, DEC2),
      disc_lin_W)

    label = jnp.concatenate(
        [jnp.ones((B, 1), f32), jnp.zeros((B, 1), f32)], axis=0
    )
    return (mu, lvs, rec, pred, label)
